# baseline (speedup 1.0000x reference)
import functools

import jax
import jax.numpy as jnp
from jax import lax
from jax.experimental import pallas as pl
from jax.experimental.pallas import tpu as pltpu
from jax.experimental.pallas import tpu_sc as plsc

VOCAB = 32000
HIDDEN = 4096
BATCH = 4
SEQ = 4096
NTOK = BATCH * SEQ
NC = 2
NS = 16
NW = NC * NS
PER_W = NTOK // NW
CHUNK = 4
NCHUNK = PER_W // CHUNK
GROUP = 4
NGROUP = NCHUNK // GROUP
NBUF = 4
NSLOT = 3

_mesh = plsc.VectorSubcoreMesh(core_axis_name="c", subcore_axis_name="s")


@functools.partial(
    pl.kernel,
    out_type=jax.ShapeDtypeStruct((NTOK, HIDDEN), jnp.float32),
    mesh=_mesh,
    scratch_types=[
        pltpu.VMEM((NCHUNK, CHUNK), jnp.int32),
        [pltpu.VMEM((CHUNK, HIDDEN), jnp.float32) for _ in range(NBUF)],
        pltpu.VMEM_SHARED((NS, NSLOT, CHUNK, HIDDEN), jnp.float32),
        [pltpu.SemaphoreType.DMA for _ in range(NBUF)],
        [pltpu.SemaphoreType.DMA for _ in range(NSLOT)],
        [pltpu.SemaphoreType.DMA for _ in range(NSLOT)],
        pltpu.SemaphoreType.DMA,
    ],
)
def _emb_lookup(idx_hbm, table_hbm, out_hbm, idx_v, bufs, shared,
                gsems, psems, dsems, ssem):
    cid = lax.axis_index("c")
    sid = lax.axis_index("s")
    wid = sid * NC + cid
    base = wid * PER_W

    pltpu.sync_copy(idx_hbm.at[wid], idx_v)

    def gather_start(c, b):
        pltpu.async_copy(table_hbm.at[idx_v.at[c]], bufs[b], gsems[b])

    def gather_wait(c, b):
        pltpu.make_async_copy(table_hbm.at[idx_v.at[c]], bufs[b], gsems[b]).wait()

    def push_start(c, s):
        pltpu.async_copy(bufs[s], shared.at[sid, s], psems[s])

    def push_wait(c, s):
        pltpu.make_async_copy(bufs[s], shared.at[sid, s], psems[s]).wait()

    def dma_start(c, s):
        pltpu.async_copy(shared.at[sid, s],
                         out_hbm.at[pl.ds(base + c * CHUNK, CHUNK)], dsems[s])

    def dma_wait(c, s):
        pltpu.make_async_copy(
            shared.at[sid, s],
            out_hbm.at[pl.ds(base + c * CHUNK, CHUNK)], dsems[s]).wait()

    def store_start(c):
        pltpu.async_copy(
            bufs[GROUP - 1],
            out_hbm.at[pl.ds(base + c * CHUNK, CHUNK)], ssem)

    def store_wait(c):
        pltpu.make_async_copy(
            bufs[GROUP - 1],
            out_hbm.at[pl.ds(base + c * CHUNK, CHUNK)], ssem).wait()

    for b in range(NBUF):
        gather_start(b, b)

    def step(i, carry):
        for lane in range(GROUP):
            c = i * GROUP + lane
            prev = (lane - 1) % GROUP
            gather_wait(c, lane)

            if lane == 0:
                @pl.when(c >= 1)
                def _retire_prev_store():
                    store_wait(c - 1)
            else:
                @pl.when(c >= 1)
                def _retire_prev_push():
                    push_wait(c - 1, prev)
                    dma_start(c - 1, prev)

            if lane < NSLOT:
                @pl.when(c >= GROUP)
                def _free_slot():
                    dma_wait(c - GROUP, lane)

                push_start(c, lane)
            else:
                store_start(c)

            @pl.when(jnp.logical_and(c >= 1, c + GROUP - 1 < NCHUNK))
            def _refill():
                gather_start(c + GROUP - 1, prev)

        return carry

    lax.fori_loop(0, NGROUP, step, 0)

    store_wait(NCHUNK - 1)
    for s in range(NSLOT):
        dma_wait(NCHUNK - GROUP + s, s)


def kernel(input_args, embed_tokens_weight):
    idx = input_args.reshape(NW, NCHUNK, CHUNK).astype(jnp.int32)
    out = _emb_lookup(idx, embed_tokens_weight)
    return out.reshape(BATCH, SEQ, HIDDEN)


# speedup vs baseline: 1.0028x; 1.0028x over previous
import functools

import jax
import jax.numpy as jnp
from jax import lax
from jax.experimental import pallas as pl
from jax.experimental.pallas import tpu as pltpu
from jax.experimental.pallas import tpu_sc as plsc

VOCAB = 32000
HIDDEN = 4096
BATCH = 4
SEQ = 4096
NTOK = BATCH * SEQ
NC = 2
NS = 16
NW = NC * NS
PER_W = NTOK // NW
CHUNK = 4
NCHUNK = PER_W // CHUNK
NBUF = 3
NSLOT = 3

_mesh = plsc.VectorSubcoreMesh(core_axis_name="c", subcore_axis_name="s")


@functools.partial(
    pl.kernel,
    out_type=jax.ShapeDtypeStruct((NTOK, HIDDEN), jnp.float32),
    mesh=_mesh,
    scratch_types=[
        pltpu.VMEM((NCHUNK, CHUNK), jnp.int32),
        [pltpu.VMEM((CHUNK, HIDDEN), jnp.float32) for _ in range(NBUF)],
        pltpu.VMEM_SHARED((NS, NSLOT, CHUNK, HIDDEN), jnp.float32),
        [pltpu.SemaphoreType.DMA for _ in range(NBUF)],
        [pltpu.SemaphoreType.DMA for _ in range(NBUF)],
        [pltpu.SemaphoreType.DMA for _ in range(NSLOT)],
    ],
)
def _emb_lookup(idx_hbm, table_hbm, out_hbm, idx_v, bufs, shared,
                gsems, psems, dsems):
    cid = lax.axis_index("c")
    sid = lax.axis_index("s")
    wid = sid * NC + cid
    base = wid * PER_W

    pltpu.sync_copy(idx_hbm.at[wid], idx_v)

    def gather_start(c, b):
        pltpu.async_copy(table_hbm.at[idx_v.at[c]], bufs[b], gsems[b])

    def gather_wait(c, b):
        pltpu.make_async_copy(table_hbm.at[idx_v.at[c]], bufs[b], gsems[b]).wait()

    def push_start(c, b):
        pltpu.async_copy(bufs[b], shared.at[sid, c % NSLOT], psems[b])

    def push_wait(c, b):
        pltpu.make_async_copy(
            bufs[b], shared.at[sid, c % NSLOT], psems[b]).wait()

    def dma_start(c, r):
        pltpu.async_copy(shared.at[sid, c % NSLOT],
                         out_hbm.at[pl.ds(base + c * CHUNK, CHUNK)], dsems[r])

    def dma_wait(c, r):
        pltpu.make_async_copy(
            shared.at[sid, c % NSLOT],
            out_hbm.at[pl.ds(base + c * CHUNK, CHUNK)], dsems[r]).wait()

    for b in range(NBUF):
        gather_start(b, b)

    def step(i, carry):
        for lane in range(NBUF):
            c = i * NBUF + lane
            b = lane
            pb = (lane - 1) % NBUF
            gather_wait(c, b)

            @pl.when(c >= 1)
            def _push_prev():
                push_wait(c - 1, pb)
                dma_start(c - 1, pb)

            @pl.when(c >= NSLOT)
            def _free_slot():
                dma_wait(c - NSLOT, b)

            push_start(c, b)

            @pl.when(jnp.logical_and(c >= 1, c + NBUF - 1 < NCHUNK))
            def _refill():
                gather_start(c + NBUF - 1, pb)

        return carry

    lax.fori_loop(0, NCHUNK // NBUF, step, 0)

    for c in range(NBUF * (NCHUNK // NBUF), NCHUNK):
        b = c % NBUF
        pb = (b - 1) % NBUF
        gather_wait(c, b)
        push_wait(c - 1, pb)
        dma_start(c - 1, pb)
        dma_wait(c - NSLOT, b)
        push_start(c, b)

    last = NCHUNK - 1
    push_wait(last, last % NBUF)
    dma_start(last, last % NSLOT)
    for k in range(NSLOT - 1, -1, -1):
        dma_wait(last - k, (last - k) % NSLOT)


def kernel(input_args, embed_tokens_weight):
    idx = input_args.reshape(NW, NCHUNK, CHUNK).astype(jnp.int32)
    out = _emb_lookup(idx, embed_tokens_weight)
    return out.reshape(BATCH, SEQ, HIDDEN)
